# baseline (speedup 1.0000x reference)
import functools

import jax
import jax.numpy as jnp
from jax.experimental import pallas as pl
from jax.experimental.pallas import tpu as pltpu

TOP_K = 8
BT = 512


def _router_block(x_ref, w1_ref, b1_ref, w2_ref, b2_ref, w3_ref, b3_ref,
                  idx_ref, wgt_ref, scr_ref):
    i = pl.program_id(0)
    par = jax.lax.rem(i, 2)

    x = x_ref[...]
    h = jnp.dot(x, w1_ref[...], preferred_element_type=jnp.float32)
    h = jnp.maximum(h + b1_ref[...], 0.0)
    h = jnp.dot(h, w2_ref[...], preferred_element_type=jnp.float32)
    h = jnp.maximum(h + b2_ref[...], 0.0)
    sc = jnp.dot(h, w3_ref[...], preferred_element_type=jnp.float32)
    sc = sc + b3_ref[...]
    scr_ref[1 - par] = sc

    s = scr_ref[par]
    num_e = s.shape[-1]
    flane = jax.lax.broadcasted_iota(jnp.int32, s.shape, 1).astype(jnp.float32)
    vals = []
    idxs = []
    for _ in range(TOP_K):
        mx = jnp.max(s, axis=1, keepdims=True)
        imf = jnp.min(jnp.where(s == mx, flane, jnp.float32(num_e)),
                      axis=1, keepdims=True)
        vals.append(mx)
        idxs.append(imf)
        s = jnp.where(flane == imf, -jnp.inf, s)
    v = jnp.concatenate(vals, axis=1)
    i32 = jnp.concatenate(idxs, axis=1).astype(jnp.int32)
    e = jnp.exp(v - v[:, :1])
    w = e / jnp.sum(e, axis=1, keepdims=True)
    idx_ref[...] = i32
    wgt_ref[...] = w


@jax.jit
def _run(x, w1, b1, w2, b2, w3, b3):
    b, d = x.shape
    h = w1.shape[1]
    e = w3.shape[1]
    nb = b // BT
    grid = (nb + 1,)
    last = nb - 1
    return pl.pallas_call(
        _router_block,
        grid=grid,
        in_specs=[
            pl.BlockSpec((BT, d), lambda i: (jnp.minimum(i, last), 0)),
            pl.BlockSpec((d, h), lambda i: (0, 0)),
            pl.BlockSpec((1, h), lambda i: (0, 0)),
            pl.BlockSpec((h, h), lambda i: (0, 0)),
            pl.BlockSpec((1, h), lambda i: (0, 0)),
            pl.BlockSpec((h, e), lambda i: (0, 0)),
            pl.BlockSpec((1, e), lambda i: (0, 0)),
        ],
        out_specs=[
            pl.BlockSpec((BT, TOP_K), lambda i: (jnp.maximum(i - 1, 0), 0)),
            pl.BlockSpec((BT, TOP_K), lambda i: (jnp.maximum(i - 1, 0), 0)),
        ],
        out_shape=[
            jax.ShapeDtypeStruct((b, TOP_K), jnp.int32),
            jax.ShapeDtypeStruct((b, TOP_K), jnp.float32),
        ],
        scratch_shapes=[pltpu.VMEM((2, BT, e), jnp.float32)],
    )(x, w1, b1, w2, b2, w3, b3)


def kernel(prompt_embedding, W1, b1, W2, b2, W3, b3):
    idx, wgt = _run(prompt_embedding.astype(jnp.float32), W1,
                    b1.reshape(1, -1), W2, b2.reshape(1, -1), W3,
                    b3.reshape(1, -1))
    return idx, wgt, idx[:, 0]
